# baseline (speedup 1.0000x reference)
import jax
import jax.numpy as jnp
from jax import lax
from jax.experimental import pallas as pl
from jax.experimental.pallas import tpu as pltpu
from jax.experimental.pallas import tpu_sc as plsc

N = 4096 * 4096
NC = 2
NS = 16
L = 16
NW = NC * NS
PT = N // NW
C = 4096
NCHUNK = PT // C
FLUSH_AT = C - 2 * L
T1N = 0x7F800000
DUMP = 0x7F800001
DMASK = 0x3FFFF
T2N = DUMP + DMASK + C
MASK = 0x7FFFFFFF
MINI32 = -2147483648
LSTRIDE = PT + C

_mesh = plsc.VectorSubcoreMesh(core_axis_name="c", subcore_axis_name="s")


def _scatter_body(bits_hbm, t1_hbm, bits_v, idx_v, pay_v, sem):
    wid = lax.axis_index("s") * NC + lax.axis_index("c")
    iota = lax.iota(jnp.int32, L)

    def chunk(ch, carry):
        base = wid * PT + ch * C
        pltpu.sync_copy(bits_hbm.at[pl.ds(base, C)], bits_v)

        def vec(i, c2):
            b = bits_v[pl.ds(i * L, L)]
            cb = jnp.where(b == MINI32, 0, b)
            sign = jnp.where(cb < 0, 1, 0)
            idx_v[pl.ds(i * L, L)] = cb & MASK
            pay_v[pl.ds(i * L, L)] = (iota + (base + i * L)) * 2 + sign
            return c2

        lax.fori_loop(0, C // L, vec, 0)
        pltpu.async_copy(pay_v, t1_hbm.at[idx_v], sem).wait()
        return carry

    lax.fori_loop(0, NCHUNK, chunk, 0)


def _classify_body(bits_hbm, t1_hbm, t2_hbm, acc_out, cnt_out, lidx_hbm,
                   lpay_hbm, bits_v, m_v, g_v, sidx_v, spay_v, acc_v, tmp_v,
                   sem):
    wid = lax.axis_index("s") * NC + lax.axis_index("c")
    iota = lax.iota(jnp.int32, L)
    ones = iota * 0 + 1
    zeros = iota * 0
    acc_v[pl.ds(0, L)] = zeros
    lb = wid * LSTRIDE

    def ini(j, c2):
        sidx_v[pl.ds(j * L, L)] = iota + (DUMP + j * L)
        spay_v[pl.ds(j * L, L)] = zeros - 1
        return c2

    lax.fori_loop(0, C // L, ini, 0)

    def flush(pf, fl):
        def sent(j, c2):
            gl = iota + j * L
            keep = gl < pf
            si = sidx_v[pl.ds(j * L, L)]
            sp = spay_v[pl.ds(j * L, L)]
            dump = DUMP + ((gl + fl * 7) & DMASK)
            sidx_v[pl.ds(j * L, L)] = jnp.where(keep, si, dump)
            spay_v[pl.ds(j * L, L)] = jnp.where(keep, sp, zeros - 1)
            return c2

        lax.fori_loop(0, C // L, sent, 0)
        pltpu.async_copy(spay_v, t2_hbm.at[sidx_v], sem).wait()
        pltpu.sync_copy(sidx_v, lidx_hbm.at[pl.ds(lb + fl * C, C)])
        pltpu.sync_copy(spay_v, lpay_hbm.at[pl.ds(lb + fl * C, C)])

    def chunk(ch, carry):
        base = wid * PT + ch * C
        pltpu.sync_copy(bits_hbm.at[pl.ds(base, C)], bits_v)

        def vec1(i, c2):
            b = bits_v[pl.ds(i * L, L)]
            cb = jnp.where(b == MINI32, 0, b)
            m_v[pl.ds(i * L, L)] = cb & MASK
            return c2

        lax.fori_loop(0, C // L, vec1, 0)
        pltpu.async_copy(t1_hbm.at[m_v], g_v, sem).wait()

        def vec2(i, carry2):
            pf, fl = carry2
            b = bits_v[pl.ds(i * L, L)]
            cb = jnp.where(b == MINI32, 0, b)
            sign = jnp.where(cb < 0, 1, 0)
            pay = (iota + (base + i * L)) * 2 + sign
            g = g_v[pl.ds(i * L, L)]
            acc_v[pl.ds(0, L)] = acc_v[pl.ds(0, L)] + jnp.where(
                g == pay, ones, zeros)
            loser = jnp.where((g & 1) != sign, ones, zeros)
            m = m_v[pl.ds(i * L, L)]
            l0 = loser[0]
            p1 = l0
            p2 = p1 + loser[1]
            p3 = p2 + loser[2]
            p4 = p3 + loser[3]
            p5 = p4 + loser[4]
            p6 = p5 + loser[5]
            p7 = p6 + loser[6]
            p8 = p7 + loser[7]
            p9 = p8 + loser[8]
            p10 = p9 + loser[9]
            p11 = p10 + loser[10]
            p12 = p11 + loser[11]
            p13 = p12 + loser[12]
            p14 = p13 + loser[13]
            p15 = p14 + loser[14]
            ls = p15 + loser[15]
            pres = (0, p1, p2, p3, p4, p5, p6, p7, p8, p9, p10, p11, p12,
                    p13, p14, p15)

            @pl.when(ls > 0)
            def _():
                for lane in range(L):
                    @pl.when(loser[lane] > 0)
                    def _():
                        dst = pf + pres[lane]
                        sidx_v[pl.ds(dst, L)] = zeros + m[lane]
                        spay_v[pl.ds(dst, L)] = zeros + pay[lane]

            pf = pf + ls

            @pl.when(pf > FLUSH_AT)
            def _():
                flush(pf, fl)

            fl = jnp.where(pf > FLUSH_AT, fl + 1, fl)
            pf = jnp.where(pf > FLUSH_AT, 0, pf)
            return (pf, fl)

        return lax.fori_loop(0, C // L, vec2, carry)

    pf, fl = lax.fori_loop(0, NCHUNK, chunk, (0, 0))
    flush(pf, fl)
    fl = fl + 1
    tmp_v[pl.ds(0, L)] = zeros + fl * C
    pltpu.sync_copy(tmp_v, cnt_out.at[wid])
    pltpu.sync_copy(acc_v, acc_out.at[wid])


def _resolve_body(t2_hbm, lidx_hbm, lpay_hbm, cnt_hbm, out_hbm, idx_v,
                  pay_v, g2_v, acc_v, tmp_v, sem):
    wid = lax.axis_index("s") * NC + lax.axis_index("c")
    iota = lax.iota(jnp.int32, L)
    ones = iota * 0 + 1
    zeros = iota * 0
    acc_v[pl.ds(0, L)] = zeros
    lb = wid * LSTRIDE
    pltpu.sync_copy(cnt_hbm.at[wid], tmp_v)
    cntv = tmp_v[pl.ds(0, L)]
    nch = cntv[0] // C

    def chunk(j, c2):
        pltpu.sync_copy(lidx_hbm.at[pl.ds(lb + j * C, C)], idx_v)
        pltpu.sync_copy(lpay_hbm.at[pl.ds(lb + j * C, C)], pay_v)
        pltpu.async_copy(t2_hbm.at[idx_v], g2_v, sem).wait()

        def cmp(k, c3):
            p = pay_v[pl.ds(k * L, L)]
            hit = (p != -1) & (g2_v[pl.ds(k * L, L)] == p)
            acc_v[pl.ds(0, L)] = acc_v[pl.ds(0, L)] + jnp.where(
                hit, ones, zeros)
            return c3

        lax.fori_loop(0, C // L, cmp, 0)
        return c2

    lax.fori_loop(0, nch, chunk, 0)
    pltpu.sync_copy(acc_v, out_hbm.at[wid])


_scatter_call = pl.kernel(
    _scatter_body,
    out_type=jax.ShapeDtypeStruct((T1N,), jnp.int32),
    mesh=_mesh,
    scratch_types=[
        pltpu.VMEM((C,), jnp.int32),
        pltpu.VMEM((C,), jnp.int32),
        pltpu.VMEM((C,), jnp.int32),
        pltpu.SemaphoreType.DMA,
    ],
)

_classify_call = pl.kernel(
    _classify_body,
    out_type=[
        jax.ShapeDtypeStruct((T2N,), jnp.int32),
        jax.ShapeDtypeStruct((NW, L), jnp.int32),
        jax.ShapeDtypeStruct((NW, L), jnp.int32),
        jax.ShapeDtypeStruct((NW * LSTRIDE,), jnp.int32),
        jax.ShapeDtypeStruct((NW * LSTRIDE,), jnp.int32),
    ],
    mesh=_mesh,
    scratch_types=[
        pltpu.VMEM((C,), jnp.int32),
        pltpu.VMEM((C,), jnp.int32),
        pltpu.VMEM((C,), jnp.int32),
        pltpu.VMEM((C,), jnp.int32),
        pltpu.VMEM((C,), jnp.int32),
        pltpu.VMEM((L,), jnp.int32),
        pltpu.VMEM((L,), jnp.int32),
        pltpu.SemaphoreType.DMA,
    ],
)

_resolve_call = pl.kernel(
    _resolve_body,
    out_type=jax.ShapeDtypeStruct((NW, L), jnp.int32),
    mesh=_mesh,
    scratch_types=[
        pltpu.VMEM((C,), jnp.int32),
        pltpu.VMEM((C,), jnp.int32),
        pltpu.VMEM((C,), jnp.int32),
        pltpu.VMEM((L,), jnp.int32),
        pltpu.VMEM((L,), jnp.int32),
        pltpu.SemaphoreType.DMA,
    ],
)


def kernel(prediction):
    bits = lax.bitcast_convert_type(prediction.reshape(-1), jnp.int32)
    t1 = _scatter_call(bits)
    t2, acc1, cnt, lidx, lpay = _classify_call(bits, t1)
    acc2 = _resolve_call(t2, lidx, lpay, cnt)
    num_unique = acc1.sum() + acc2.sum()
    return (jnp.int32(N) - num_unique).astype(jnp.float32)


# speedup vs baseline: 1.0195x; 1.0195x over previous
import jax
import jax.numpy as jnp
from jax import lax
from jax.experimental import pallas as pl
from jax.experimental.pallas import tpu as pltpu
from jax.experimental.pallas import tpu_sc as plsc

N = 4096 * 4096
NC = 2
NS = 16
L = 16
NW = NC * NS
PT = N // NW
C = 2048
NCHUNK = PT // C
FLUSH_AT = C - 2 * L
T1N = 0x7F800000
DUMP = 0x7F800001
DMASK = 0x3FFFF
T2N = DUMP + DMASK + C
MASK = 0x7FFFFFFF
MINI32 = -2147483648
LSTRIDE = PT + C

_mesh = plsc.VectorSubcoreMesh(core_axis_name="c", subcore_axis_name="s")


def _scatter_body(bits_hbm, t1_hbm, bits_v, idx_v, pay_v, sem):
    wid = lax.axis_index("s") * NC + lax.axis_index("c")
    iota = lax.iota(jnp.int32, L)

    def chunk(ch, carry):
        base = wid * PT + ch * C
        pltpu.sync_copy(bits_hbm.at[pl.ds(base, C)], bits_v)

        def vec(i, c2):
            b = bits_v[pl.ds(i * L, L)]
            cb = jnp.where(b == MINI32, 0, b)
            sign = jnp.where(cb < 0, 1, 0)
            idx_v[pl.ds(i * L, L)] = cb & MASK
            pay_v[pl.ds(i * L, L)] = (iota + (base + i * L)) * 2 + sign
            return c2

        lax.fori_loop(0, C // L, vec, 0)
        pltpu.async_copy(pay_v, t1_hbm.at[idx_v], sem).wait()
        return carry

    lax.fori_loop(0, NCHUNK, chunk, 0)


def _classify_body(bits_hbm, t1_hbm, t2_hbm, acc_out, cnt_out, lidx_hbm,
                   lpay_hbm, bits_v, m_v, g_v, bits_b, m_b, g_b, sidx_v,
                   spay_v, acc_v, tmp_v, sem):
    wid = lax.axis_index("s") * NC + lax.axis_index("c")
    iota = lax.iota(jnp.int32, L)
    ones = iota * 0 + 1
    zeros = iota * 0
    acc_v[pl.ds(0, L)] = zeros
    lb = wid * LSTRIDE

    def ini(j, c2):
        sidx_v[pl.ds(j * L, L)] = iota + (DUMP + j * L)
        spay_v[pl.ds(j * L, L)] = zeros - 1
        return c2

    lax.fori_loop(0, C // L, ini, 0)

    def flush(pf, fl):
        def sent(j, c2):
            gl = iota + j * L
            keep = gl < pf
            si = sidx_v[pl.ds(j * L, L)]
            sp = spay_v[pl.ds(j * L, L)]
            dump = DUMP + ((gl + fl * 7) & DMASK)
            sidx_v[pl.ds(j * L, L)] = jnp.where(keep, si, dump)
            spay_v[pl.ds(j * L, L)] = jnp.where(keep, sp, zeros - 1)
            return c2

        lax.fori_loop(0, C // L, sent, 0)
        pltpu.async_copy(spay_v, t2_hbm.at[sidx_v], sem).wait()
        pltpu.sync_copy(sidx_v, lidx_hbm.at[pl.ds(lb + fl * C, C)])
        pltpu.sync_copy(spay_v, lpay_hbm.at[pl.ds(lb + fl * C, C)])

    def load_half(base, bits_ref, m_ref, g_ref):
        pltpu.sync_copy(bits_hbm.at[pl.ds(base, C)], bits_ref)

        def vec1(i, c2):
            b = bits_ref[pl.ds(i * L, L)]
            cb = jnp.where(b == MINI32, 0, b)
            m_ref[pl.ds(i * L, L)] = cb & MASK
            return c2

        lax.fori_loop(0, C // L, vec1, 0)
        return pltpu.async_copy(t1_hbm.at[m_ref], g_ref, sem)

    def classify_half(base, bits_ref, g_ref, carry):
        def vec2(i, carry2):
            pf, fl = carry2
            b = bits_ref[pl.ds(i * L, L)]
            cb = jnp.where(b == MINI32, 0, b)
            sign = jnp.where(cb < 0, 1, 0)
            pay = (iota + (base + i * L)) * 2 + sign
            g = g_ref[pl.ds(i * L, L)]
            acc_v[pl.ds(0, L)] = acc_v[pl.ds(0, L)] + jnp.where(
                g == pay, ones, zeros)
            loser = jnp.where((g & 1) != sign, ones, zeros)
            m = cb & MASK
            l0 = loser[0]
            p1 = l0
            p2 = p1 + loser[1]
            p3 = p2 + loser[2]
            p4 = p3 + loser[3]
            p5 = p4 + loser[4]
            p6 = p5 + loser[5]
            p7 = p6 + loser[6]
            p8 = p7 + loser[7]
            p9 = p8 + loser[8]
            p10 = p9 + loser[9]
            p11 = p10 + loser[10]
            p12 = p11 + loser[11]
            p13 = p12 + loser[12]
            p14 = p13 + loser[13]
            p15 = p14 + loser[14]
            ls = p15 + loser[15]
            pres = (0, p1, p2, p3, p4, p5, p6, p7, p8, p9, p10, p11, p12,
                    p13, p14, p15)

            @pl.when(ls > 0)
            def _():
                for lane in range(L):
                    @pl.when(loser[lane] > 0)
                    def _():
                        dst = pf + pres[lane]
                        sidx_v[pl.ds(dst, L)] = zeros + m[lane]
                        spay_v[pl.ds(dst, L)] = zeros + pay[lane]

            pf = pf + ls

            @pl.when(pf > FLUSH_AT)
            def _():
                flush(pf, fl)

            fl = jnp.where(pf > FLUSH_AT, fl + 1, fl)
            pf = jnp.where(pf > FLUSH_AT, 0, pf)
            return (pf, fl)

        return lax.fori_loop(0, C // L, vec2, carry)

    def chunk(ch, carry):
        base_a = wid * PT + (2 * ch) * C
        base_b = base_a + C
        da = load_half(base_a, bits_v, m_v, g_v)
        db = load_half(base_b, bits_b, m_b, g_b)
        da.wait()
        carry = classify_half(base_a, bits_v, g_v, carry)
        db.wait()
        carry = classify_half(base_b, bits_b, g_b, carry)
        return carry

    pf, fl = lax.fori_loop(0, NCHUNK // 2, chunk, (0, 0))
    flush(pf, fl)
    fl = fl + 1
    tmp_v[pl.ds(0, L)] = zeros + fl * C
    pltpu.sync_copy(tmp_v, cnt_out.at[wid])
    pltpu.sync_copy(acc_v, acc_out.at[wid])


def _resolve_body(t2_hbm, lidx_hbm, lpay_hbm, cnt_hbm, out_hbm, idx_v,
                  pay_v, g2_v, acc_v, tmp_v, sem):
    wid = lax.axis_index("s") * NC + lax.axis_index("c")
    iota = lax.iota(jnp.int32, L)
    ones = iota * 0 + 1
    zeros = iota * 0
    acc_v[pl.ds(0, L)] = zeros
    lb = wid * LSTRIDE
    pltpu.sync_copy(cnt_hbm.at[wid], tmp_v)
    cntv = tmp_v[pl.ds(0, L)]
    nch = cntv[0] // C

    def chunk(j, c2):
        pltpu.sync_copy(lidx_hbm.at[pl.ds(lb + j * C, C)], idx_v)
        pltpu.sync_copy(lpay_hbm.at[pl.ds(lb + j * C, C)], pay_v)
        pltpu.async_copy(t2_hbm.at[idx_v], g2_v, sem).wait()

        def cmp(k, c3):
            p = pay_v[pl.ds(k * L, L)]
            hit = (p != -1) & (g2_v[pl.ds(k * L, L)] == p)
            acc_v[pl.ds(0, L)] = acc_v[pl.ds(0, L)] + jnp.where(
                hit, ones, zeros)
            return c3

        lax.fori_loop(0, C // L, cmp, 0)
        return c2

    lax.fori_loop(0, nch, chunk, 0)
    pltpu.sync_copy(acc_v, out_hbm.at[wid])


_scatter_call = pl.kernel(
    _scatter_body,
    out_type=jax.ShapeDtypeStruct((T1N,), jnp.int32),
    mesh=_mesh,
    scratch_types=[
        pltpu.VMEM((C,), jnp.int32),
        pltpu.VMEM((C,), jnp.int32),
        pltpu.VMEM((C,), jnp.int32),
        pltpu.SemaphoreType.DMA,
    ],
)

_classify_call = pl.kernel(
    _classify_body,
    out_type=[
        jax.ShapeDtypeStruct((T2N,), jnp.int32),
        jax.ShapeDtypeStruct((NW, L), jnp.int32),
        jax.ShapeDtypeStruct((NW, L), jnp.int32),
        jax.ShapeDtypeStruct((NW * LSTRIDE,), jnp.int32),
        jax.ShapeDtypeStruct((NW * LSTRIDE,), jnp.int32),
    ],
    mesh=_mesh,
    scratch_types=[
        pltpu.VMEM((C,), jnp.int32),
        pltpu.VMEM((C,), jnp.int32),
        pltpu.VMEM((C,), jnp.int32),
        pltpu.VMEM((C,), jnp.int32),
        pltpu.VMEM((C,), jnp.int32),
        pltpu.VMEM((C,), jnp.int32),
        pltpu.VMEM((C,), jnp.int32),
        pltpu.VMEM((C,), jnp.int32),
        pltpu.VMEM((L,), jnp.int32),
        pltpu.VMEM((L,), jnp.int32),
        pltpu.SemaphoreType.DMA,
    ],
)

_resolve_call = pl.kernel(
    _resolve_body,
    out_type=jax.ShapeDtypeStruct((NW, L), jnp.int32),
    mesh=_mesh,
    scratch_types=[
        pltpu.VMEM((C,), jnp.int32),
        pltpu.VMEM((C,), jnp.int32),
        pltpu.VMEM((C,), jnp.int32),
        pltpu.VMEM((L,), jnp.int32),
        pltpu.VMEM((L,), jnp.int32),
        pltpu.SemaphoreType.DMA,
    ],
)


def kernel(prediction):
    bits = lax.bitcast_convert_type(prediction.reshape(-1), jnp.int32)
    t1 = _scatter_call(bits)
    t2, acc1, cnt, lidx, lpay = _classify_call(bits, t1)
    acc2 = _resolve_call(t2, lidx, lpay, cnt)
    num_unique = acc1.sum() + acc2.sum()
    return (jnp.int32(N) - num_unique).astype(jnp.float32)


# speedup vs baseline: 1.0330x; 1.0133x over previous
import jax
import jax.numpy as jnp
from jax import lax
from jax.experimental import pallas as pl
from jax.experimental.pallas import tpu as pltpu
from jax.experimental.pallas import tpu_sc as plsc

N = 4096 * 4096
NC = 2
NS = 16
L = 16
NW = NC * NS
PT = N // NW
C = 2048
NCHUNK = PT // C
FLUSH_AT = C - 2 * L
T1N = 0x7F800000
DUMP = 0x7F800001
DMASK = 0x3FFFF
T2N = DUMP + DMASK + C
MASK = 0x7FFFFFFF
MINI32 = -2147483648
LSTRIDE = PT + C

_mesh = plsc.VectorSubcoreMesh(core_axis_name="c", subcore_axis_name="s")


def _scatter_body(bits_hbm, t1_hbm, m_out, pay_out, bits_v, idx_v, pay_v,
                  sem):
    wid = lax.axis_index("s") * NC + lax.axis_index("c")
    iota = lax.iota(jnp.int32, L)

    def chunk(ch, carry):
        base = wid * PT + ch * C
        pltpu.sync_copy(bits_hbm.at[pl.ds(base, C)], bits_v)

        def vec(i, c2):
            b = bits_v[pl.ds(i * L, L)]
            cb = jnp.where(b == MINI32, 0, b)
            sign = jnp.where(cb < 0, 1, 0)
            idx_v[pl.ds(i * L, L)] = cb & MASK
            pay_v[pl.ds(i * L, L)] = (iota + (base + i * L)) * 2 + sign
            return c2

        lax.fori_loop(0, C // L, vec, 0)
        d = pltpu.async_copy(pay_v, t1_hbm.at[idx_v], sem)
        pltpu.sync_copy(idx_v, m_out.at[pl.ds(base, C)])
        pltpu.sync_copy(pay_v, pay_out.at[pl.ds(base, C)])
        d.wait()
        return carry

    lax.fori_loop(0, NCHUNK, chunk, 0)


def _classify_body(m_hbm, pay_hbm, t1_hbm, t2_hbm, acc_out, cnt_out,
                   lidx_hbm, lpay_hbm, pay_v, m_v, g_v, pay_b, m_b, g_b,
                   sidx_v, spay_v, acc_v, tmp_v, sem):
    wid = lax.axis_index("s") * NC + lax.axis_index("c")
    iota = lax.iota(jnp.int32, L)
    ones = iota * 0 + 1
    zeros = iota * 0
    acc_v[pl.ds(0, L)] = zeros
    lb = wid * LSTRIDE

    def ini(j, c2):
        sidx_v[pl.ds(j * L, L)] = iota + (DUMP + j * L)
        spay_v[pl.ds(j * L, L)] = zeros - 1
        return c2

    lax.fori_loop(0, C // L, ini, 0)

    def flush(pf, fl):
        def sent(j, c2):
            gl = iota + j * L
            keep = gl < pf
            si = sidx_v[pl.ds(j * L, L)]
            sp = spay_v[pl.ds(j * L, L)]
            dump = DUMP + ((gl + fl * 7) & DMASK)
            sidx_v[pl.ds(j * L, L)] = jnp.where(keep, si, dump)
            spay_v[pl.ds(j * L, L)] = jnp.where(keep, sp, zeros - 1)
            return c2

        lax.fori_loop(0, C // L, sent, 0)
        pltpu.async_copy(spay_v, t2_hbm.at[sidx_v], sem).wait()
        pltpu.sync_copy(sidx_v, lidx_hbm.at[pl.ds(lb + fl * C, C)])
        pltpu.sync_copy(spay_v, lpay_hbm.at[pl.ds(lb + fl * C, C)])

    def load_half(base, pay_ref, m_ref, g_ref):
        pltpu.sync_copy(m_hbm.at[pl.ds(base, C)], m_ref)
        pltpu.sync_copy(pay_hbm.at[pl.ds(base, C)], pay_ref)
        return pltpu.async_copy(t1_hbm.at[m_ref], g_ref, sem)

    def classify_half(base, pay_ref, m_ref, g_ref, carry):
        def vec2(i, carry2):
            pf, fl = carry2
            pay = pay_ref[pl.ds(i * L, L)]
            sign = pay & 1
            g = g_ref[pl.ds(i * L, L)]
            acc_v[pl.ds(0, L)] = acc_v[pl.ds(0, L)] + jnp.where(
                g == pay, ones, zeros)
            loser = jnp.where((g & 1) != sign, ones, zeros)
            m = m_ref[pl.ds(i * L, L)]
            l0 = loser[0]
            p1 = l0
            p2 = p1 + loser[1]
            p3 = p2 + loser[2]
            p4 = p3 + loser[3]
            p5 = p4 + loser[4]
            p6 = p5 + loser[5]
            p7 = p6 + loser[6]
            p8 = p7 + loser[7]
            p9 = p8 + loser[8]
            p10 = p9 + loser[9]
            p11 = p10 + loser[10]
            p12 = p11 + loser[11]
            p13 = p12 + loser[12]
            p14 = p13 + loser[13]
            p15 = p14 + loser[14]
            ls = p15 + loser[15]
            pres = (0, p1, p2, p3, p4, p5, p6, p7, p8, p9, p10, p11, p12,
                    p13, p14, p15)

            @pl.when(ls > 0)
            def _():
                for lane in range(L):
                    @pl.when(loser[lane] > 0)
                    def _():
                        dst = pf + pres[lane]
                        sidx_v[pl.ds(dst, L)] = zeros + m[lane]
                        spay_v[pl.ds(dst, L)] = zeros + pay[lane]

            pf = pf + ls

            @pl.when(pf > FLUSH_AT)
            def _():
                flush(pf, fl)

            fl = jnp.where(pf > FLUSH_AT, fl + 1, fl)
            pf = jnp.where(pf > FLUSH_AT, 0, pf)
            return (pf, fl)

        return lax.fori_loop(0, C // L, vec2, carry)

    def chunk(ch, carry):
        base_a = wid * PT + (2 * ch) * C
        base_b = base_a + C
        da = load_half(base_a, pay_v, m_v, g_v)
        db = load_half(base_b, pay_b, m_b, g_b)
        da.wait()
        carry = classify_half(base_a, pay_v, m_v, g_v, carry)
        db.wait()
        carry = classify_half(base_b, pay_b, m_b, g_b, carry)
        return carry

    pf, fl = lax.fori_loop(0, NCHUNK // 2, chunk, (0, 0))
    flush(pf, fl)
    fl = fl + 1
    tmp_v[pl.ds(0, L)] = zeros + fl * C
    pltpu.sync_copy(tmp_v, cnt_out.at[wid])
    pltpu.sync_copy(acc_v, acc_out.at[wid])


def _resolve_body(t2_hbm, lidx_hbm, lpay_hbm, cnt_hbm, out_hbm, idx_v,
                  pay_v, g2_v, acc_v, tmp_v, sem):
    wid = lax.axis_index("s") * NC + lax.axis_index("c")
    iota = lax.iota(jnp.int32, L)
    ones = iota * 0 + 1
    zeros = iota * 0
    acc_v[pl.ds(0, L)] = zeros
    lb = wid * LSTRIDE
    pltpu.sync_copy(cnt_hbm.at[wid], tmp_v)
    cntv = tmp_v[pl.ds(0, L)]
    nch = cntv[0] // C

    def chunk(j, c2):
        pltpu.sync_copy(lidx_hbm.at[pl.ds(lb + j * C, C)], idx_v)
        pltpu.sync_copy(lpay_hbm.at[pl.ds(lb + j * C, C)], pay_v)
        pltpu.async_copy(t2_hbm.at[idx_v], g2_v, sem).wait()

        def cmp(k, c3):
            p = pay_v[pl.ds(k * L, L)]
            hit = (p != -1) & (g2_v[pl.ds(k * L, L)] == p)
            acc_v[pl.ds(0, L)] = acc_v[pl.ds(0, L)] + jnp.where(
                hit, ones, zeros)
            return c3

        lax.fori_loop(0, C // L, cmp, 0)
        return c2

    lax.fori_loop(0, nch, chunk, 0)
    pltpu.sync_copy(acc_v, out_hbm.at[wid])


_scatter_call = pl.kernel(
    _scatter_body,
    out_type=[
        jax.ShapeDtypeStruct((T1N,), jnp.int32),
        jax.ShapeDtypeStruct((N,), jnp.int32),
        jax.ShapeDtypeStruct((N,), jnp.int32),
    ],
    mesh=_mesh,
    scratch_types=[
        pltpu.VMEM((C,), jnp.int32),
        pltpu.VMEM((C,), jnp.int32),
        pltpu.VMEM((C,), jnp.int32),
        pltpu.SemaphoreType.DMA,
    ],
)

_classify_call = pl.kernel(
    _classify_body,
    out_type=[
        jax.ShapeDtypeStruct((T2N,), jnp.int32),
        jax.ShapeDtypeStruct((NW, L), jnp.int32),
        jax.ShapeDtypeStruct((NW, L), jnp.int32),
        jax.ShapeDtypeStruct((NW * LSTRIDE,), jnp.int32),
        jax.ShapeDtypeStruct((NW * LSTRIDE,), jnp.int32),
    ],
    mesh=_mesh,
    scratch_types=[
        pltpu.VMEM((C,), jnp.int32),
        pltpu.VMEM((C,), jnp.int32),
        pltpu.VMEM((C,), jnp.int32),
        pltpu.VMEM((C,), jnp.int32),
        pltpu.VMEM((C,), jnp.int32),
        pltpu.VMEM((C,), jnp.int32),
        pltpu.VMEM((C,), jnp.int32),
        pltpu.VMEM((C,), jnp.int32),
        pltpu.VMEM((L,), jnp.int32),
        pltpu.VMEM((L,), jnp.int32),
        pltpu.SemaphoreType.DMA,
    ],
)

_resolve_call = pl.kernel(
    _resolve_body,
    out_type=jax.ShapeDtypeStruct((NW, L), jnp.int32),
    mesh=_mesh,
    scratch_types=[
        pltpu.VMEM((C,), jnp.int32),
        pltpu.VMEM((C,), jnp.int32),
        pltpu.VMEM((C,), jnp.int32),
        pltpu.VMEM((L,), jnp.int32),
        pltpu.VMEM((L,), jnp.int32),
        pltpu.SemaphoreType.DMA,
    ],
)


def kernel(prediction):
    bits = lax.bitcast_convert_type(prediction.reshape(-1), jnp.int32)
    t1, m_arr, pay_arr = _scatter_call(bits)
    t2, acc1, cnt, lidx, lpay = _classify_call(m_arr, pay_arr, t1)
    acc2 = _resolve_call(t2, lidx, lpay, cnt)
    num_unique = acc1.sum() + acc2.sum()
    return (jnp.int32(N) - num_unique).astype(jnp.float32)
